# baseline (speedup 1.0000x reference)
import functools

import jax
import jax.numpy as jnp
from jax.experimental import pallas as pl
from jax.experimental.pallas import tpu as pltpu

N = 5000
NP = 5120
ROWS = NP // 128
DETS = 100
SCORE_THRESH = 0.05
IDX_BIG = 2**30
KEY_DEAD = -(2**31) + 1


def _nms_kernel(bj_ref, sj_ref, out_ref,
                x1j, y1j, x2j, y2j, aj, suppj, suppt, keyref, rmaxref):
    f32 = jnp.float32
    cx = bj_ref[0] * 1024.0
    cy = bj_ref[1] * 1024.0
    w = bj_ref[2] * 256.0 + 1.0
    h = bj_ref[3] * 256.0 + 1.0
    x1j[...] = cx - w * 0.5
    y1j[...] = cy - h * 0.5
    x2j[...] = cx + w * 0.5
    y2j[...] = cy + h * 0.5
    aj[...] = (x2j[...] - x1j[...]) * (y2j[...] - y1j[...])

    x1c = jnp.transpose(x1j[...])
    y1c = jnp.transpose(y1j[...])
    x2c = jnp.transpose(x2j[...])
    y2c = jnp.transpose(y2j[...])
    ac = jnp.transpose(aj[...])
    sc = jnp.transpose(sj_ref[...])

    lane = jax.lax.broadcasted_iota(jnp.int32, (1, 128), 1)
    suppj[...] = jnp.full((ROWS, 128), -1.0, f32)

    for r in range(ROWS):
        shp = (128, 128)
        bx1 = jnp.broadcast_to(x1c[:, r:r + 1], shp)
        by1 = jnp.broadcast_to(y1c[:, r:r + 1], shp)
        bx2 = jnp.broadcast_to(x2c[:, r:r + 1], shp)
        by2 = jnp.broadcast_to(y2c[:, r:r + 1], shp)
        ba = jnp.broadcast_to(ac[:, r:r + 1], shp)
        bs = jnp.broadcast_to(sc[:, r:r + 1], shp)

        def cval(jb):
            x1 = x1j[pl.ds(jb, 1), :]
            y1 = y1j[pl.ds(jb, 1), :]
            x2 = x2j[pl.ds(jb, 1), :]
            y2 = y2j[pl.ds(jb, 1), :]
            ar = aj[pl.ds(jb, 1), :]
            lw = jnp.minimum(bx2, x2) - jnp.maximum(bx1, x1)
            lh = jnp.minimum(by2, y2) - jnp.maximum(by1, y1)
            inter = jnp.maximum(lw, 0.0) * jnp.maximum(lh, 0.0)
            return inter - 0.5 * ((ba + ar) - inter)

        cv = cval(r)
        sj = sj_ref[pl.ds(r, 1), :]
        bii = jnp.broadcast_to(
            r * 128 + jax.lax.broadcasted_iota(jnp.int32, (128, 1), 0), shp)
        dom = (bs > sj) | ((bs == sj) & (bii < r * 128 + lane))
        m1 = jnp.where(dom, cv, -1.0)
        suppj[pl.ds(r, 1), :] = jnp.maximum(
            suppj[pl.ds(r, 1), :], jnp.max(m1, axis=0, keepdims=True))

        def j_step(jb, acc2):
            cv = cval(jb)
            sj = sj_ref[pl.ds(jb, 1), :]
            dom = bs >= sj
            m1 = jnp.where(dom, cv, -1.0)
            suppj[pl.ds(jb, 1), :] = jnp.maximum(
                suppj[pl.ds(jb, 1), :], jnp.max(m1, axis=0, keepdims=True))
            return jnp.maximum(acc2, jnp.where(dom, -1.0, cv))

        def j_body4(k, accs):
            a, b = accs
            jb = r + 1 + 4 * k
            a = j_step(jb, a)
            b = j_step(jb + 1, b)
            a = j_step(jb + 2, a)
            b = j_step(jb + 3, b)
            return a, b

        nb = ROWS - 1 - r
        acc2, acc2b = jax.lax.fori_loop(
            0, nb // 4, j_body4,
            (jnp.full(shp, -1.0, f32), jnp.full(shp, -1.0, f32)))
        for jb in range(r + 1 + 4 * (nb // 4), ROWS):
            acc2 = j_step(jb, acc2)
        acc2 = jnp.maximum(acc2, acc2b)
        suppt[:, r:r + 1] = jnp.max(acc2, axis=1, keepdims=True)

    supp = jnp.maximum(suppj[...], jnp.transpose(suppt[...]))

    sj = sj_ref[...]
    keep = (supp <= 0.0) & (sj > SCORE_THRESH)
    kbits = jax.lax.bitcast_convert_type(sj, jnp.int32)
    key = jnp.where(keep, kbits | jnp.int32(1 << 30), kbits)
    keyref[...] = key
    rmaxref[...] = jnp.max(key, axis=1, keepdims=True)
    iota_r = jax.lax.broadcasted_iota(jnp.int32, (ROWS, 1), 0)

    def t_body(t, _):
        rmax_v = rmaxref[...]
        m11 = jnp.max(rmax_v, axis=0, keepdims=True)
        rp = jnp.min(jnp.where(rmax_v == m11, iota_r, IDX_BIG))
        row = keyref[pl.ds(rp, 1), :]
        cmin = jnp.min(jnp.where(row == m11, lane, IDX_BIG),
                       axis=1, keepdims=True)
        hot = lane == cmin
        for col, ref in enumerate((x1j, y1j, x2j, y2j)):
            brow = ref[pl.ds(rp, 1), :]
            out_ref[pl.ds(t, 1), col:col + 1] = jnp.sum(
                jnp.where(hot, brow, 0.0), axis=1, keepdims=True)
        scv = jnp.where(m11 >= jnp.int32(1 << 30),
                        jax.lax.bitcast_convert_type(
                            m11 & jnp.int32(0x3FFFFFFF), jnp.float32),
                        jnp.float32(0.0))
        out_ref[pl.ds(t, 1), 4:5] = scv
        row2 = jnp.where(hot, KEY_DEAD, row)
        keyref[pl.ds(rp, 1), :] = row2
        rmaxref[pl.ds(rp, 1), 0:1] = jnp.max(row2, axis=1, keepdims=True)
        return 0

    jax.lax.fori_loop(0, DETS, t_body, 0)


@jax.jit
def kernel(boxes, scores):
    pad = NP - N
    bpad = jnp.pad(boxes, ((0, pad), (0, 0)))
    spad = jnp.pad(scores, (0, pad), constant_values=-1.0)
    bj = bpad.T.reshape(4, ROWS, 128)
    sjm = spad.reshape(ROWS, 128)

    f32 = jnp.float32
    out = pl.pallas_call(
        _nms_kernel,
        out_shape=jax.ShapeDtypeStruct((DETS, 5), f32),
        scratch_shapes=(
            [pltpu.VMEM((ROWS, 128), f32) for _ in range(5)]
            + [pltpu.VMEM((ROWS, 128), f32),
               pltpu.VMEM((128, ROWS), f32),
               pltpu.VMEM((ROWS, 128), jnp.int32),
               pltpu.VMEM((ROWS, 1), jnp.int32)]),
    )(bj, sjm)
    return out


# speedup vs baseline: 1.1098x; 1.1098x over previous
import functools

import jax
import jax.numpy as jnp
from jax.experimental import pallas as pl
from jax.experimental.pallas import tpu as pltpu

N = 5000
NP = 5120
ROWS = NP // 128
DETS = 100
SCORE_THRESH = 0.05
IDX_BIG = 2**30
KEY_DEAD = -(2**31) + 1


def _nms_kernel(bj_ref, sj_ref, out_ref,
                x1j, y1j, x2j, y2j, aj, suppj, suppt, keyref, rmaxref,
                picksref):
    f32 = jnp.float32
    cx = bj_ref[0] * 1024.0
    cy = bj_ref[1] * 1024.0
    w = bj_ref[2] * 256.0 + 1.0
    h = bj_ref[3] * 256.0 + 1.0
    x1j[...] = cx - w * 0.5
    y1j[...] = cy - h * 0.5
    x2j[...] = cx + w * 0.5
    y2j[...] = cy + h * 0.5
    aj[...] = (x2j[...] - x1j[...]) * (y2j[...] - y1j[...])

    x1c = jnp.transpose(x1j[...])
    y1c = jnp.transpose(y1j[...])
    x2c = jnp.transpose(x2j[...])
    y2c = jnp.transpose(y2j[...])
    ac = jnp.transpose(aj[...])
    sc = jnp.transpose(sj_ref[...])

    lane = jax.lax.broadcasted_iota(jnp.int32, (1, 128), 1)
    suppj[...] = jnp.full((ROWS, 128), -1.0, f32)

    for r in range(ROWS):
        shp = (128, 128)
        bx1 = jnp.broadcast_to(x1c[:, r:r + 1], shp)
        by1 = jnp.broadcast_to(y1c[:, r:r + 1], shp)
        bx2 = jnp.broadcast_to(x2c[:, r:r + 1], shp)
        by2 = jnp.broadcast_to(y2c[:, r:r + 1], shp)
        ba = jnp.broadcast_to(ac[:, r:r + 1], shp)
        bs = jnp.broadcast_to(sc[:, r:r + 1], shp)

        def cval(jb):
            x1 = x1j[pl.ds(jb, 1), :]
            y1 = y1j[pl.ds(jb, 1), :]
            x2 = x2j[pl.ds(jb, 1), :]
            y2 = y2j[pl.ds(jb, 1), :]
            ar = aj[pl.ds(jb, 1), :]
            lw = jnp.minimum(bx2, x2) - jnp.maximum(bx1, x1)
            lh = jnp.minimum(by2, y2) - jnp.maximum(by1, y1)
            inter = jnp.maximum(lw, 0.0) * jnp.maximum(lh, 0.0)
            return inter - 0.5 * ((ba + ar) - inter)

        cv = cval(r)
        sj = sj_ref[pl.ds(r, 1), :]
        bii = jnp.broadcast_to(
            r * 128 + jax.lax.broadcasted_iota(jnp.int32, (128, 1), 0), shp)
        dom = (bs > sj) | ((bs == sj) & (bii < r * 128 + lane))
        m1 = jnp.where(dom, cv, -1.0)
        suppj[pl.ds(r, 1), :] = jnp.maximum(
            suppj[pl.ds(r, 1), :], jnp.max(m1, axis=0, keepdims=True))

        def j_step(jb, acc2):
            cv = cval(jb)
            sj = sj_ref[pl.ds(jb, 1), :]
            dom = bs >= sj
            m1 = jnp.where(dom, cv, -1.0)
            suppj[pl.ds(jb, 1), :] = jnp.maximum(
                suppj[pl.ds(jb, 1), :], jnp.max(m1, axis=0, keepdims=True))
            return jnp.maximum(acc2, jnp.where(dom, -1.0, cv))

        def j_body4(k, accs):
            a, b = accs
            jb = r + 1 + 4 * k
            a = j_step(jb, a)
            b = j_step(jb + 1, b)
            a = j_step(jb + 2, a)
            b = j_step(jb + 3, b)
            return a, b

        nb = ROWS - 1 - r
        acc2, acc2b = jax.lax.fori_loop(
            0, nb // 4, j_body4,
            (jnp.full(shp, -1.0, f32), jnp.full(shp, -1.0, f32)))
        for jb in range(r + 1 + 4 * (nb // 4), ROWS):
            acc2 = j_step(jb, acc2)
        acc2 = jnp.maximum(acc2, acc2b)
        suppt[:, r:r + 1] = jnp.max(acc2, axis=1, keepdims=True)

    supp = jnp.maximum(suppj[...], jnp.transpose(suppt[...]))

    sj = sj_ref[...]
    keep = (supp <= 0.0) & (sj > SCORE_THRESH)
    kbits = jax.lax.bitcast_convert_type(sj, jnp.int32)
    key = jnp.where(keep, kbits | jnp.int32(1 << 30), kbits)
    keyref[...] = key
    rmaxref[...] = jnp.max(key, axis=1, keepdims=True)
    iota_r = jax.lax.broadcasted_iota(jnp.int32, (ROWS, 1), 0)

    def t_body(t, _):
        key_v = keyref[...]
        rmax_v = rmaxref[...]
        m11 = jnp.max(rmax_v, axis=0, keepdims=True)
        rmin = jnp.min(jnp.where(rmax_v == m11, iota_r, IDX_BIG),
                       axis=0, keepdims=True)
        rhot = iota_r == rmin
        row = jnp.sum(jnp.where(rhot, key_v, 0), axis=0, keepdims=True)
        cmin = jnp.min(jnp.where(row == m11, lane, IDX_BIG),
                       axis=1, keepdims=True)
        hot = lane == cmin
        picksref[pl.ds(t, 1), 0:1] = rmin * 128 + cmin
        scv = jnp.where(m11 >= jnp.int32(1 << 30),
                        jax.lax.bitcast_convert_type(
                            m11 & jnp.int32(0x3FFFFFFF), jnp.float32),
                        jnp.float32(0.0))
        out_ref[pl.ds(t, 1), 4:5] = scv
        keyref[...] = jnp.where(rhot & hot, KEY_DEAD, key_v)
        rm2 = jnp.max(jnp.where(hot, KEY_DEAD, row), axis=1, keepdims=True)
        rmaxref[...] = jnp.where(rhot, rm2, rmax_v)
        return 0

    jax.lax.fori_loop(0, DETS, t_body, 0)

    picks = picksref[...]
    oh_r = (picks // 128 == jax.lax.broadcasted_iota(
        jnp.int32, (DETS, ROWS), 1)).astype(f32)
    oh_c = (picks % 128 == jax.lax.broadcasted_iota(
        jnp.int32, (DETS, 128), 1)).astype(f32)
    for col, ref in enumerate((x1j, y1j, x2j, y2j)):
        rowsv = jax.lax.dot_general(
            oh_r, ref[...], (((1,), (0,)), ((), ())),
            preferred_element_type=f32)
        out_ref[:, col:col + 1] = jnp.sum(
            rowsv * oh_c, axis=1, keepdims=True)


@jax.jit
def kernel(boxes, scores):
    pad = NP - N
    bpad = jnp.pad(boxes, ((0, pad), (0, 0)))
    spad = jnp.pad(scores, (0, pad), constant_values=-1.0)
    bj = bpad.T.reshape(4, ROWS, 128)
    sjm = spad.reshape(ROWS, 128)

    f32 = jnp.float32
    out = pl.pallas_call(
        _nms_kernel,
        out_shape=jax.ShapeDtypeStruct((DETS, 5), f32),
        scratch_shapes=(
            [pltpu.VMEM((ROWS, 128), f32) for _ in range(5)]
            + [pltpu.VMEM((ROWS, 128), f32),
               pltpu.VMEM((128, ROWS), f32),
               pltpu.VMEM((ROWS, 128), jnp.int32),
               pltpu.VMEM((ROWS, 1), jnp.int32),
               pltpu.VMEM((DETS, 1), jnp.int32)]),
    )(bj, sjm)
    return out


# speedup vs baseline: 1.1105x; 1.0007x over previous
import functools

import jax
import jax.numpy as jnp
from jax.experimental import pallas as pl
from jax.experimental.pallas import tpu as pltpu

N = 5000
NP = 5120
ROWS = NP // 128
DETS = 100
SCORE_THRESH = 0.05
IDX_BIG = 2**30
KEY_DEAD = -(2**31) + 1


def _nms_kernel(bj_ref, sj_ref, out_ref,
                x1j, y1j, x2j, y2j, aj, suppj, suppt, keyref, rmaxref,
                picksref):
    f32 = jnp.float32
    cx = bj_ref[0] * 1024.0
    cy = bj_ref[1] * 1024.0
    w = bj_ref[2] * 256.0 + 1.0
    h = bj_ref[3] * 256.0 + 1.0
    x1j[...] = cx - w * 0.5
    y1j[...] = cy - h * 0.5
    x2j[...] = cx + w * 0.5
    y2j[...] = cy + h * 0.5
    aj[...] = (x2j[...] - x1j[...]) * (y2j[...] - y1j[...])

    x1c = jnp.transpose(x1j[...])
    y1c = jnp.transpose(y1j[...])
    x2c = jnp.transpose(x2j[...])
    y2c = jnp.transpose(y2j[...])
    ac = jnp.transpose(aj[...])
    sc = jnp.transpose(sj_ref[...])

    lane = jax.lax.broadcasted_iota(jnp.int32, (1, 128), 1)
    suppj[...] = jnp.full((ROWS, 128), -1.0, f32)

    for r in range(ROWS):
        shp = (128, 128)
        bx1 = jnp.broadcast_to(x1c[:, r:r + 1], shp)
        by1 = jnp.broadcast_to(y1c[:, r:r + 1], shp)
        bx2 = jnp.broadcast_to(x2c[:, r:r + 1], shp)
        by2 = jnp.broadcast_to(y2c[:, r:r + 1], shp)
        ba = jnp.broadcast_to(ac[:, r:r + 1], shp)
        bs = jnp.broadcast_to(sc[:, r:r + 1], shp)

        def cval(jb):
            x1 = x1j[pl.ds(jb, 1), :]
            y1 = y1j[pl.ds(jb, 1), :]
            x2 = x2j[pl.ds(jb, 1), :]
            y2 = y2j[pl.ds(jb, 1), :]
            ar = aj[pl.ds(jb, 1), :]
            lw = jnp.minimum(bx2, x2) - jnp.maximum(bx1, x1)
            lh = jnp.minimum(by2, y2) - jnp.maximum(by1, y1)
            inter = jnp.maximum(lw, 0.0) * jnp.maximum(lh, 0.0)
            return inter - 0.5 * ((ba + ar) - inter)

        cv = cval(r)
        sj = sj_ref[pl.ds(r, 1), :]
        bii = jnp.broadcast_to(
            r * 128 + jax.lax.broadcasted_iota(jnp.int32, (128, 1), 0), shp)
        dom = (bs > sj) | ((bs == sj) & (bii < r * 128 + lane))
        m1 = jnp.where(dom, cv, -1.0)
        suppj[pl.ds(r, 1), :] = jnp.maximum(
            suppj[pl.ds(r, 1), :], jnp.max(m1, axis=0, keepdims=True))

        def j_step(jb, acc2):
            cv = cval(jb)
            sj = sj_ref[pl.ds(jb, 1), :]
            dom = bs >= sj
            m1 = jnp.where(dom, cv, -1.0)
            suppj[pl.ds(jb, 1), :] = jnp.maximum(
                suppj[pl.ds(jb, 1), :], jnp.max(m1, axis=0, keepdims=True))
            return jnp.maximum(acc2, jnp.where(dom, -1.0, cv))

        def j_body4(k, accs):
            a, b = accs
            jb = r + 1 + 4 * k
            a = j_step(jb, a)
            b = j_step(jb + 1, b)
            a = j_step(jb + 2, a)
            b = j_step(jb + 3, b)
            return a, b

        nb = ROWS - 1 - r
        acc2, acc2b = jax.lax.fori_loop(
            0, nb // 4, j_body4,
            (jnp.full(shp, -1.0, f32), jnp.full(shp, -1.0, f32)))
        for jb in range(r + 1 + 4 * (nb // 4), ROWS):
            acc2 = j_step(jb, acc2)
        acc2 = jnp.maximum(acc2, acc2b)
        suppt[:, r:r + 1] = jnp.max(acc2, axis=1, keepdims=True)

    supp = jnp.maximum(suppj[...], jnp.transpose(suppt[...]))

    sj = sj_ref[...]
    keep = (supp <= 0.0) & (sj > SCORE_THRESH)
    kbits = jax.lax.bitcast_convert_type(sj, jnp.int32)
    key = jnp.where(keep, kbits | jnp.int32(1 << 30), kbits)
    keyref[...] = key
    rmaxref[...] = jnp.max(key, axis=1, keepdims=True)
    iota_r = jax.lax.broadcasted_iota(jnp.int32, (ROWS, 1), 0)

    def t_body(t, _):
        key_v = keyref[...]
        rmax_v = rmaxref[...]
        m11 = jnp.max(rmax_v, axis=0, keepdims=True)
        rmin = jnp.min(jnp.where(rmax_v == m11, iota_r, IDX_BIG),
                       axis=0, keepdims=True)
        rhot = iota_r == rmin
        row = jnp.sum(jnp.where(rhot, key_v, 0), axis=0, keepdims=True)
        cmin = jnp.min(jnp.where(row == m11, lane, IDX_BIG),
                       axis=1, keepdims=True)
        hot = lane == cmin
        picksref[pl.ds(t, 1), 0:1] = rmin * 128 + cmin
        scv = jnp.where(m11 >= jnp.int32(1 << 30),
                        jax.lax.bitcast_convert_type(
                            m11 & jnp.int32(0x3FFFFFFF), jnp.float32),
                        jnp.float32(0.0))
        out_ref[pl.ds(t, 1), 4:5] = scv
        keyref[...] = jnp.where(rhot & hot, KEY_DEAD, key_v)
        rm2 = jnp.max(jnp.where(hot, KEY_DEAD, row), axis=1, keepdims=True)
        rmaxref[...] = jnp.where(rhot, rm2, rmax_v)
        return 0

    def t_body2(k, _):
        t_body(2 * k, 0)
        t_body(2 * k + 1, 0)
        return 0

    jax.lax.fori_loop(0, DETS // 2, t_body2, 0)

    picks = picksref[...]
    oh_r = (picks // 128 == jax.lax.broadcasted_iota(
        jnp.int32, (DETS, ROWS), 1)).astype(f32)
    oh_c = (picks % 128 == jax.lax.broadcasted_iota(
        jnp.int32, (DETS, 128), 1)).astype(f32)
    for col, ref in enumerate((x1j, y1j, x2j, y2j)):
        rowsv = jax.lax.dot_general(
            oh_r, ref[...], (((1,), (0,)), ((), ())),
            preferred_element_type=f32)
        out_ref[:, col:col + 1] = jnp.sum(
            rowsv * oh_c, axis=1, keepdims=True)


@jax.jit
def kernel(boxes, scores):
    pad = NP - N
    bpad = jnp.pad(boxes, ((0, pad), (0, 0)))
    spad = jnp.pad(scores, (0, pad), constant_values=-1.0)
    bj = bpad.T.reshape(4, ROWS, 128)
    sjm = spad.reshape(ROWS, 128)

    f32 = jnp.float32
    out = pl.pallas_call(
        _nms_kernel,
        out_shape=jax.ShapeDtypeStruct((DETS, 5), f32),
        scratch_shapes=(
            [pltpu.VMEM((ROWS, 128), f32) for _ in range(5)]
            + [pltpu.VMEM((ROWS, 128), f32),
               pltpu.VMEM((128, ROWS), f32),
               pltpu.VMEM((ROWS, 128), jnp.int32),
               pltpu.VMEM((ROWS, 1), jnp.int32),
               pltpu.VMEM((DETS, 1), jnp.int32)]),
    )(bj, sjm)
    return out


# speedup vs baseline: 1.8027x; 1.6232x over previous
import functools

import jax
import jax.numpy as jnp
from jax.experimental import pallas as pl
from jax.experimental.pallas import tpu as pltpu

N = 5000
NP = 5120
ROWS = NP // 128
DETS = 100
SCORE_THRESH = 0.05
IDX_BIG = 2**30
KEY_DEAD = -(2**31) + 1


def _nms_kernel(bj_ref, sj_ref, out_ref,
                x1j, y1j, x2j, y2j, aj, suppj, suppt):
    f32 = jnp.float32
    cx = bj_ref[0] * 1024.0
    cy = bj_ref[1] * 1024.0
    w = bj_ref[2] * 256.0 + 1.0
    h = bj_ref[3] * 256.0 + 1.0
    x1j[...] = cx - w * 0.5
    y1j[...] = cy - h * 0.5
    x2j[...] = cx + w * 0.5
    y2j[...] = cy + h * 0.5
    aj[...] = (x2j[...] - x1j[...]) * (y2j[...] - y1j[...])

    x1c = jnp.transpose(x1j[...])
    y1c = jnp.transpose(y1j[...])
    x2c = jnp.transpose(x2j[...])
    y2c = jnp.transpose(y2j[...])
    ac = jnp.transpose(aj[...])
    sc = jnp.transpose(sj_ref[...])

    lane = jax.lax.broadcasted_iota(jnp.int32, (1, 128), 1)
    suppj[...] = jnp.full((ROWS, 128), -1.0, f32)

    for r in range(ROWS):
        shp = (128, 128)
        bx1 = jnp.broadcast_to(x1c[:, r:r + 1], shp)
        by1 = jnp.broadcast_to(y1c[:, r:r + 1], shp)
        bx2 = jnp.broadcast_to(x2c[:, r:r + 1], shp)
        by2 = jnp.broadcast_to(y2c[:, r:r + 1], shp)
        ba = jnp.broadcast_to(ac[:, r:r + 1], shp)
        bs = jnp.broadcast_to(sc[:, r:r + 1], shp)

        def cval(jb):
            x1 = x1j[pl.ds(jb, 1), :]
            y1 = y1j[pl.ds(jb, 1), :]
            x2 = x2j[pl.ds(jb, 1), :]
            y2 = y2j[pl.ds(jb, 1), :]
            ar = aj[pl.ds(jb, 1), :]
            lw = jnp.minimum(bx2, x2) - jnp.maximum(bx1, x1)
            lh = jnp.minimum(by2, y2) - jnp.maximum(by1, y1)
            inter = jnp.maximum(lw, 0.0) * jnp.maximum(lh, 0.0)
            return inter - 0.5 * ((ba + ar) - inter)

        cv = cval(r)
        sj = sj_ref[pl.ds(r, 1), :]
        bii = jnp.broadcast_to(
            r * 128 + jax.lax.broadcasted_iota(jnp.int32, (128, 1), 0), shp)
        dom = (bs > sj) | ((bs == sj) & (bii < r * 128 + lane))
        m1 = jnp.where(dom, cv, -1.0)
        suppj[pl.ds(r, 1), :] = jnp.maximum(
            suppj[pl.ds(r, 1), :], jnp.max(m1, axis=0, keepdims=True))

        def j_step(jb, acc2):
            cv = cval(jb)
            sj = sj_ref[pl.ds(jb, 1), :]
            dom = bs >= sj
            m1 = jnp.where(dom, cv, -1.0)
            suppj[pl.ds(jb, 1), :] = jnp.maximum(
                suppj[pl.ds(jb, 1), :], jnp.max(m1, axis=0, keepdims=True))
            return jnp.maximum(acc2, jnp.where(dom, -1.0, cv))

        def j_body4(k, accs):
            a, b = accs
            jb = r + 1 + 4 * k
            a = j_step(jb, a)
            b = j_step(jb + 1, b)
            a = j_step(jb + 2, a)
            b = j_step(jb + 3, b)
            return a, b

        nb = ROWS - 1 - r
        acc2, acc2b = jax.lax.fori_loop(
            0, nb // 4, j_body4,
            (jnp.full(shp, -1.0, f32), jnp.full(shp, -1.0, f32)))
        for jb in range(r + 1 + 4 * (nb // 4), ROWS):
            acc2 = j_step(jb, acc2)
        acc2 = jnp.maximum(acc2, acc2b)
        suppt[:, r:r + 1] = jnp.max(acc2, axis=1, keepdims=True)

    supp = jnp.maximum(suppj[...], jnp.transpose(suppt[...]))

    sj = sj_ref[...]
    keep = (supp <= 0.0) & (sj > SCORE_THRESH)
    kbits = jax.lax.bitcast_convert_type(sj, jnp.int32)
    key = jnp.where(keep, kbits | jnp.int32(1 << 30), kbits)

    one11 = jnp.ones((1, 1), jnp.int32)
    lo = -one11
    hi = jnp.full((1, 1), 0x7F7FFFFF, jnp.int32)
    for _ in range(31):
        mid11 = lo + jax.lax.shift_right_arithmetic(hi - lo, 1)
        cnt = jnp.sum(jnp.where(key > mid11, 1.0, 0.0), keepdims=True)
        cond = cnt <= float(DETS - 1)
        hi = jnp.where(cond, mid11, hi)
        lo = jnp.where(cond, lo, mid11)
    theta = hi

    gmask = key > theta
    emask = key == theta
    gm = jnp.where(gmask, 1.0, 0.0)
    em = jnp.where(emask, 1.0, 0.0)

    lane_f = jax.lax.broadcasted_iota(jnp.int32, (1, 128), 1)
    tril_s = jnp.where(
        jax.lax.broadcasted_iota(jnp.int32, (128, 128), 0)
        < jax.lax.broadcasted_iota(jnp.int32, (128, 128), 1), 1.0, 0.0)
    rows_g = jnp.sum(gm, axis=1, keepdims=True)
    rows_e = jnp.sum(em, axis=1, keepdims=True)
    tril_r = jnp.where(
        jax.lax.broadcasted_iota(jnp.int32, (ROWS, ROWS), 0)
        < jax.lax.broadcasted_iota(jnp.int32, (ROWS, ROWS), 1), 1.0, 0.0)
    rpre_g = jax.lax.dot_general(
        jnp.transpose(rows_g), tril_r, (((1,), (0,)), ((), ())),
        preferred_element_type=jnp.float32)
    rpre_e = jax.lax.dot_general(
        jnp.transpose(rows_e), tril_r, (((1,), (0,)), ((), ())),
        preferred_element_type=jnp.float32)
    lpre_g = jax.lax.dot_general(
        gm, tril_s, (((1,), (0,)), ((), ())),
        preferred_element_type=jnp.float32)
    lpre_e = jax.lax.dot_general(
        em, tril_s, (((1,), (0,)), ((), ())),
        preferred_element_type=jnp.float32)
    pos_g = jnp.where(gmask, jnp.transpose(rpre_g) + lpre_g, -1.0)
    pos_e = jnp.where(emask, jnp.transpose(rpre_e) + lpre_e, -1.0)

    outscore = jnp.where(keep, sj, 0.0)
    keepf = jnp.where(keep, 1.0, 0.0)
    onesrow = jnp.ones((1, 128), f32)
    lane_ff = lane_f.astype(f32)

    cg = jnp.zeros((8, 128), f32)
    ce = jnp.zeros((8, 128), f32)
    for r in range(ROWS):
        vals = jnp.concatenate(
            [x1j[pl.ds(r, 1), :], y1j[pl.ds(r, 1), :],
             x2j[pl.ds(r, 1), :], y2j[pl.ds(r, 1), :],
             outscore[r:r + 1, :], keepf[r:r + 1, :],
             sj[r:r + 1, :], onesrow], axis=0)
        pgc = jnp.transpose(pos_g[r:r + 1, :])
        pec = jnp.transpose(pos_e[r:r + 1, :])
        cg = cg + jax.lax.dot_general(
            vals, jnp.where(pgc == lane_ff, 1.0, 0.0),
            (((1,), (0,)), ((), ())), preferred_element_type=f32)
        ce = ce + jax.lax.dot_general(
            vals, jnp.where(pec == lane_ff, 1.0, 0.0),
            (((1,), (0,)), ((), ())), preferred_element_type=f32)

    validg = cg[7:8, :]
    kf = cg[5:6, :]
    sr = cg[6:7, :]
    kfT = jnp.transpose(kf)
    srT = jnp.transpose(sr)
    vT = jnp.transpose(validg)
    iot = lane_ff
    iotT = jnp.transpose(iot)
    beats = (vT > 0.0) & (
        (kfT > kf) | ((kfT == kf) & ((srT > sr) | ((srT == sr)
                                                   & (iotT < iot)))))
    rank_g = jnp.sum(jnp.where(beats, 1.0, 0.0), axis=0, keepdims=True)
    g11 = jnp.sum(gm, keepdims=True)

    iota_t = jax.lax.broadcasted_iota(jnp.int32, (DETS, 1), 0).astype(f32)
    ohg = jnp.where((rank_g == iota_t) & (validg > 0.0), 1.0, 0.0)
    rank_e = g11 + lane_ff
    ohe = jnp.where((rank_e == iota_t) & (ce[7:8, :] > 0.0), 1.0, 0.0)
    outg = jax.lax.dot_general(
        ohg, jnp.transpose(cg), (((1,), (0,)), ((), ())),
        preferred_element_type=f32)
    oute = jax.lax.dot_general(
        ohe, jnp.transpose(ce), (((1,), (0,)), ((), ())),
        preferred_element_type=f32)
    outfull = outg + oute
    out_ref[...] = outfull[:, 0:5]


@jax.jit
def kernel(boxes, scores):
    pad = NP - N
    bpad = jnp.pad(boxes, ((0, pad), (0, 0)))
    spad = jnp.pad(scores, (0, pad), constant_values=-1.0)
    bj = bpad.T.reshape(4, ROWS, 128)
    sjm = spad.reshape(ROWS, 128)

    f32 = jnp.float32
    out = pl.pallas_call(
        _nms_kernel,
        out_shape=jax.ShapeDtypeStruct((DETS, 5), f32),
        scratch_shapes=(
            [pltpu.VMEM((ROWS, 128), f32) for _ in range(5)]
            + [pltpu.VMEM((ROWS, 128), f32),
               pltpu.VMEM((128, ROWS), f32),
]),
    )(bj, sjm)
    return out


# speedup vs baseline: 1.8145x; 1.0065x over previous
import functools

import jax
import jax.numpy as jnp
from jax.experimental import pallas as pl
from jax.experimental.pallas import tpu as pltpu

N = 5000
NP = 5120
ROWS = NP // 128
DETS = 100
SCORE_THRESH = 0.05
IDX_BIG = 2**30
KEY_DEAD = -(2**31) + 1


def _nms_kernel(bj_ref, sj_ref, out_ref,
                x1j, y1j, x2j, y2j, aj, suppj, suppt):
    f32 = jnp.float32
    cx = bj_ref[0] * 1024.0
    cy = bj_ref[1] * 1024.0
    w = bj_ref[2] * 256.0 + 1.0
    h = bj_ref[3] * 256.0 + 1.0
    x1j[...] = cx - w * 0.5
    y1j[...] = cy - h * 0.5
    x2j[...] = cx + w * 0.5
    y2j[...] = cy + h * 0.5
    aj[...] = (x2j[...] - x1j[...]) * (y2j[...] - y1j[...])

    x1c = jnp.transpose(x1j[...])
    y1c = jnp.transpose(y1j[...])
    x2c = jnp.transpose(x2j[...])
    y2c = jnp.transpose(y2j[...])
    ac = jnp.transpose(aj[...])
    sc = jnp.transpose(sj_ref[...])

    lane = jax.lax.broadcasted_iota(jnp.int32, (1, 128), 1)
    suppj[...] = jnp.full((ROWS, 128), -1.0, f32)

    for r in range(ROWS):
        shp = (128, 128)
        bx1 = jnp.broadcast_to(x1c[:, r:r + 1], shp)
        by1 = jnp.broadcast_to(y1c[:, r:r + 1], shp)
        bx2 = jnp.broadcast_to(x2c[:, r:r + 1], shp)
        by2 = jnp.broadcast_to(y2c[:, r:r + 1], shp)
        ba = jnp.broadcast_to(ac[:, r:r + 1], shp)
        bs = jnp.broadcast_to(sc[:, r:r + 1], shp)

        def cval(jb):
            x1 = x1j[pl.ds(jb, 1), :]
            y1 = y1j[pl.ds(jb, 1), :]
            x2 = x2j[pl.ds(jb, 1), :]
            y2 = y2j[pl.ds(jb, 1), :]
            ar = aj[pl.ds(jb, 1), :]
            lw = jnp.minimum(bx2, x2) - jnp.maximum(bx1, x1)
            lh = jnp.minimum(by2, y2) - jnp.maximum(by1, y1)
            inter = jnp.maximum(lw, 0.0) * jnp.maximum(lh, 0.0)
            return inter - 0.5 * ((ba + ar) - inter)

        cv = cval(r)
        sj = sj_ref[pl.ds(r, 1), :]
        bii = jnp.broadcast_to(
            r * 128 + jax.lax.broadcasted_iota(jnp.int32, (128, 1), 0), shp)
        dom = (bs > sj) | ((bs == sj) & (bii < r * 128 + lane))
        m1 = jnp.where(dom, cv, -1.0)
        suppj[pl.ds(r, 1), :] = jnp.maximum(
            suppj[pl.ds(r, 1), :], jnp.max(m1, axis=0, keepdims=True))

        def j_step(jb, acc2):
            cv = cval(jb)
            sj = sj_ref[pl.ds(jb, 1), :]
            dom = bs >= sj
            m1 = jnp.where(dom, cv, -1.0)
            suppj[pl.ds(jb, 1), :] = jnp.maximum(
                suppj[pl.ds(jb, 1), :], jnp.max(m1, axis=0, keepdims=True))
            return jnp.maximum(acc2, jnp.where(dom, -1.0, cv))

        def j_body8(k, accs):
            a, b = accs
            jb = r + 1 + 8 * k
            for u in range(0, 8, 2):
                a = j_step(jb + u, a)
                b = j_step(jb + u + 1, b)
            return a, b

        nb = ROWS - 1 - r
        acc2, acc2b = jax.lax.fori_loop(
            0, nb // 8, j_body8,
            (jnp.full(shp, -1.0, f32), jnp.full(shp, -1.0, f32)))
        for jb in range(r + 1 + 8 * (nb // 8), ROWS):
            acc2 = j_step(jb, acc2)
        acc2 = jnp.maximum(acc2, acc2b)
        suppt[:, r:r + 1] = jnp.max(acc2, axis=1, keepdims=True)

    supp = jnp.maximum(suppj[...], jnp.transpose(suppt[...]))

    sj = sj_ref[...]
    keep = (supp <= 0.0) & (sj > SCORE_THRESH)
    kbits = jax.lax.bitcast_convert_type(sj, jnp.int32)
    key = jnp.where(keep, kbits | jnp.int32(1 << 30), kbits)

    one11 = jnp.ones((1, 1), jnp.int32)
    lo = -one11
    hi = jnp.full((1, 1), 0x7F7FFFFF, jnp.int32)
    for _ in range(31):
        mid11 = lo + jax.lax.shift_right_arithmetic(hi - lo, 1)
        cnt = jnp.sum(jnp.where(key > mid11, 1.0, 0.0), keepdims=True)
        cond = cnt <= float(DETS - 1)
        hi = jnp.where(cond, mid11, hi)
        lo = jnp.where(cond, lo, mid11)
    theta = hi

    gmask = key > theta
    emask = key == theta
    gm = jnp.where(gmask, 1.0, 0.0)
    em = jnp.where(emask, 1.0, 0.0)

    lane_f = jax.lax.broadcasted_iota(jnp.int32, (1, 128), 1)
    tril_s = jnp.where(
        jax.lax.broadcasted_iota(jnp.int32, (128, 128), 0)
        < jax.lax.broadcasted_iota(jnp.int32, (128, 128), 1), 1.0, 0.0)
    rows_g = jnp.sum(gm, axis=1, keepdims=True)
    rows_e = jnp.sum(em, axis=1, keepdims=True)
    tril_r = jnp.where(
        jax.lax.broadcasted_iota(jnp.int32, (ROWS, ROWS), 0)
        < jax.lax.broadcasted_iota(jnp.int32, (ROWS, ROWS), 1), 1.0, 0.0)
    rpre_g = jax.lax.dot_general(
        jnp.transpose(rows_g), tril_r, (((1,), (0,)), ((), ())),
        preferred_element_type=jnp.float32,
        precision=jax.lax.Precision.HIGHEST)
    rpre_e = jax.lax.dot_general(
        jnp.transpose(rows_e), tril_r, (((1,), (0,)), ((), ())),
        preferred_element_type=jnp.float32,
        precision=jax.lax.Precision.HIGHEST)
    lpre_g = jax.lax.dot_general(
        gm, tril_s, (((1,), (0,)), ((), ())),
        preferred_element_type=jnp.float32,
        precision=jax.lax.Precision.HIGHEST)
    lpre_e = jax.lax.dot_general(
        em, tril_s, (((1,), (0,)), ((), ())),
        preferred_element_type=jnp.float32,
        precision=jax.lax.Precision.HIGHEST)
    pos_g = jnp.where(gmask, jnp.transpose(rpre_g) + lpre_g, -1.0)
    pos_e = jnp.where(emask, jnp.transpose(rpre_e) + lpre_e, -1.0)

    outscore = jnp.where(keep, sj, 0.0)
    keepf = jnp.where(keep, 1.0, 0.0)
    onesrow = jnp.ones((1, 128), f32)
    lane_ff = lane_f.astype(f32)

    cg = jnp.zeros((8, 128), f32)
    ce = jnp.zeros((8, 128), f32)
    for r in range(ROWS):
        vals = jnp.concatenate(
            [x1j[pl.ds(r, 1), :], y1j[pl.ds(r, 1), :],
             x2j[pl.ds(r, 1), :], y2j[pl.ds(r, 1), :],
             outscore[r:r + 1, :], keepf[r:r + 1, :],
             sj[r:r + 1, :], onesrow], axis=0)
        pgc = jnp.transpose(pos_g[r:r + 1, :])
        pec = jnp.transpose(pos_e[r:r + 1, :])
        cg = cg + jax.lax.dot_general(
            vals, jnp.where(pgc == lane_ff, 1.0, 0.0),
            (((1,), (0,)), ((), ())), preferred_element_type=f32,
        precision=jax.lax.Precision.HIGHEST)
        ce = ce + jax.lax.dot_general(
            vals, jnp.where(pec == lane_ff, 1.0, 0.0),
            (((1,), (0,)), ((), ())), preferred_element_type=f32,
        precision=jax.lax.Precision.HIGHEST)

    validg = cg[7:8, :]
    kf = cg[5:6, :]
    sr = cg[6:7, :]
    kfT = jnp.transpose(kf)
    srT = jnp.transpose(sr)
    vT = jnp.transpose(validg)
    iot = lane_ff
    iotT = jnp.transpose(iot)
    beats = (vT > 0.0) & (
        (kfT > kf) | ((kfT == kf) & ((srT > sr) | ((srT == sr)
                                                   & (iotT < iot)))))
    rank_g = jnp.sum(jnp.where(beats, 1.0, 0.0), axis=0, keepdims=True)
    g11 = jnp.sum(gm, keepdims=True)

    iota_t = jax.lax.broadcasted_iota(jnp.int32, (DETS, 1), 0).astype(f32)
    ohg = jnp.where((rank_g == iota_t) & (validg > 0.0), 1.0, 0.0)
    rank_e = g11 + lane_ff
    ohe = jnp.where((rank_e == iota_t) & (ce[7:8, :] > 0.0), 1.0, 0.0)
    outg = jax.lax.dot_general(
        ohg, jnp.transpose(cg), (((1,), (0,)), ((), ())),
        preferred_element_type=f32,
        precision=jax.lax.Precision.HIGHEST)
    oute = jax.lax.dot_general(
        ohe, jnp.transpose(ce), (((1,), (0,)), ((), ())),
        preferred_element_type=f32,
        precision=jax.lax.Precision.HIGHEST)
    outfull = outg + oute
    out_ref[...] = outfull[:, 0:5]


@jax.jit
def kernel(boxes, scores):
    pad = NP - N
    bpad = jnp.pad(boxes, ((0, pad), (0, 0)))
    spad = jnp.pad(scores, (0, pad), constant_values=-1.0)
    bj = bpad.T.reshape(4, ROWS, 128)
    sjm = spad.reshape(ROWS, 128)

    f32 = jnp.float32
    out = pl.pallas_call(
        _nms_kernel,
        out_shape=jax.ShapeDtypeStruct((DETS, 5), f32),
        scratch_shapes=(
            [pltpu.VMEM((ROWS, 128), f32) for _ in range(5)]
            + [pltpu.VMEM((ROWS, 128), f32),
               pltpu.VMEM((128, ROWS), f32),
]),
    )(bj, sjm)
    return out


# speedup vs baseline: 1.8194x; 1.0027x over previous
import functools

import jax
import jax.numpy as jnp
from jax.experimental import pallas as pl
from jax.experimental.pallas import tpu as pltpu

N = 5000
NP = 5120
ROWS = NP // 128
DETS = 100
SCORE_THRESH = 0.05
IDX_BIG = 2**30
KEY_DEAD = -(2**31) + 1


def _nms_kernel(bj_ref, sj_ref, out_ref,
                x1j, y1j, x2j, y2j, aj, suppj, suppt):
    f32 = jnp.float32
    cx = bj_ref[0] * 1024.0
    cy = bj_ref[1] * 1024.0
    w = bj_ref[2] * 256.0 + 1.0
    h = bj_ref[3] * 256.0 + 1.0
    x1j[...] = cx - w * 0.5
    y1j[...] = cy - h * 0.5
    x2j[...] = cx + w * 0.5
    y2j[...] = cy + h * 0.5
    aj[...] = (x2j[...] - x1j[...]) * (y2j[...] - y1j[...])

    x1c = jnp.transpose(x1j[...])
    y1c = jnp.transpose(y1j[...])
    x2c = jnp.transpose(x2j[...])
    y2c = jnp.transpose(y2j[...])
    ac = jnp.transpose(aj[...])
    sc = jnp.transpose(sj_ref[...])

    lane = jax.lax.broadcasted_iota(jnp.int32, (1, 128), 1)
    suppj[...] = jnp.full((ROWS, 128), -1.0, f32)

    for r in range(ROWS):
        shp = (128, 128)
        bx1 = jnp.broadcast_to(x1c[:, r:r + 1], shp)
        by1 = jnp.broadcast_to(y1c[:, r:r + 1], shp)
        bx2 = jnp.broadcast_to(x2c[:, r:r + 1], shp)
        by2 = jnp.broadcast_to(y2c[:, r:r + 1], shp)
        ba = jnp.broadcast_to(ac[:, r:r + 1], shp)
        bs = jnp.broadcast_to(sc[:, r:r + 1], shp)

        def cval(jb):
            x1 = x1j[pl.ds(jb, 1), :]
            y1 = y1j[pl.ds(jb, 1), :]
            x2 = x2j[pl.ds(jb, 1), :]
            y2 = y2j[pl.ds(jb, 1), :]
            ar = aj[pl.ds(jb, 1), :]
            lw = jnp.minimum(bx2, x2) - jnp.maximum(bx1, x1)
            lh = jnp.minimum(by2, y2) - jnp.maximum(by1, y1)
            inter = jnp.maximum(lw, 0.0) * jnp.maximum(lh, 0.0)
            return inter - 0.5 * ((ba + ar) - inter)

        cv = cval(r)
        sj = sj_ref[pl.ds(r, 1), :]
        bii = jnp.broadcast_to(
            r * 128 + jax.lax.broadcasted_iota(jnp.int32, (128, 1), 0), shp)
        dom = (bs > sj) | ((bs == sj) & (bii < r * 128 + lane))
        m1 = jnp.where(dom, cv, -1.0)
        suppj[pl.ds(r, 1), :] = jnp.maximum(
            suppj[pl.ds(r, 1), :], jnp.max(m1, axis=0, keepdims=True))

        def j_step(jb, acc2):
            cv = cval(jb)
            sj = sj_ref[pl.ds(jb, 1), :]
            dom = bs >= sj
            m1 = jnp.where(dom, cv, -1.0)
            suppj[pl.ds(jb, 1), :] = jnp.maximum(
                suppj[pl.ds(jb, 1), :], jnp.max(m1, axis=0, keepdims=True))
            return jnp.maximum(acc2, jnp.where(dom, -1.0, cv))

        def j_body8(k, accs):
            a, b = accs
            jb = r + 1 + 8 * k
            for u in range(0, 8, 2):
                a = j_step(jb + u, a)
                b = j_step(jb + u + 1, b)
            return a, b

        nb = ROWS - 1 - r
        acc2, acc2b = jax.lax.fori_loop(
            0, nb // 8, j_body8,
            (jnp.full(shp, -1.0, f32), jnp.full(shp, -1.0, f32)))
        for jb in range(r + 1 + 8 * (nb // 8), ROWS):
            acc2 = j_step(jb, acc2)
        acc2 = jnp.maximum(acc2, acc2b)
        suppt[:, r:r + 1] = jnp.max(acc2, axis=1, keepdims=True)

    supp = jnp.maximum(suppj[...], jnp.transpose(suppt[...]))

    sj = sj_ref[...]
    keep = (supp <= 0.0) & (sj > SCORE_THRESH)
    kbits = jax.lax.bitcast_convert_type(sj, jnp.int32)
    key = jnp.where(keep, kbits | jnp.int32(1 << 30), kbits)

    one11 = jnp.ones((1, 1), jnp.int32)
    lo = -one11
    hi = jnp.full((1, 1), 0x7F7FFFFF, jnp.int32)
    for _ in range(31):
        mid11 = lo + jax.lax.shift_right_arithmetic(hi - lo, 1)
        cnt = jnp.sum(jnp.where(key > mid11, 1.0, 0.0), keepdims=True)
        cond = cnt <= float(DETS - 1)
        hi = jnp.where(cond, mid11, hi)
        lo = jnp.where(cond, lo, mid11)
    theta = hi

    gmask = key > theta
    emask = key == theta
    gm = jnp.where(gmask, 1.0, 0.0)
    em = jnp.where(emask, 1.0, 0.0)

    lane_f = jax.lax.broadcasted_iota(jnp.int32, (1, 128), 1)
    tril_s = jnp.where(
        jax.lax.broadcasted_iota(jnp.int32, (128, 128), 0)
        < jax.lax.broadcasted_iota(jnp.int32, (128, 128), 1), 1.0, 0.0)
    rows_g = jnp.sum(gm, axis=1, keepdims=True)
    rows_e = jnp.sum(em, axis=1, keepdims=True)
    tril_r = jnp.where(
        jax.lax.broadcasted_iota(jnp.int32, (ROWS, ROWS), 0)
        < jax.lax.broadcasted_iota(jnp.int32, (ROWS, ROWS), 1), 1.0, 0.0)
    rpre_g = jax.lax.dot_general(
        jnp.transpose(rows_g), tril_r, (((1,), (0,)), ((), ())),
        preferred_element_type=jnp.float32,
        precision=jax.lax.Precision.HIGHEST)
    rpre_e = jax.lax.dot_general(
        jnp.transpose(rows_e), tril_r, (((1,), (0,)), ((), ())),
        preferred_element_type=jnp.float32,
        precision=jax.lax.Precision.HIGHEST)
    lpre_g = jax.lax.dot_general(
        gm, tril_s, (((1,), (0,)), ((), ())),
        preferred_element_type=jnp.float32,
        precision=jax.lax.Precision.HIGHEST)
    lpre_e = jax.lax.dot_general(
        em, tril_s, (((1,), (0,)), ((), ())),
        preferred_element_type=jnp.float32,
        precision=jax.lax.Precision.HIGHEST)
    pos_g = jnp.where(gmask, jnp.transpose(rpre_g) + lpre_g, -1.0)
    pos_e = jnp.where(emask, jnp.transpose(rpre_e) + lpre_e, -1.0)

    outscore = jnp.where(keep, sj, 0.0)
    keepf = jnp.where(keep, 1.0, 0.0)
    onesrow = jnp.ones((1, 128), f32)
    lane_ff = lane_f.astype(f32)

    cgs = [jnp.zeros((8, 128), f32) for _ in range(4)]
    ces = [jnp.zeros((8, 128), f32) for _ in range(4)]
    for r in range(ROWS):
        vals = jnp.concatenate(
            [x1j[pl.ds(r, 1), :], y1j[pl.ds(r, 1), :],
             x2j[pl.ds(r, 1), :], y2j[pl.ds(r, 1), :],
             outscore[r:r + 1, :], keepf[r:r + 1, :],
             sj[r:r + 1, :], onesrow], axis=0)
        pgc = jnp.transpose(pos_g[r:r + 1, :])
        pec = jnp.transpose(pos_e[r:r + 1, :])
        cgs[r % 4] = cgs[r % 4] + jax.lax.dot_general(
            vals, jnp.where(pgc == lane_ff, 1.0, 0.0),
            (((1,), (0,)), ((), ())), preferred_element_type=f32,
            precision=jax.lax.Precision.HIGHEST)
        ces[r % 4] = ces[r % 4] + jax.lax.dot_general(
            vals, jnp.where(pec == lane_ff, 1.0, 0.0),
            (((1,), (0,)), ((), ())), preferred_element_type=f32,
            precision=jax.lax.Precision.HIGHEST)

    cg = (cgs[0] + cgs[1]) + (cgs[2] + cgs[3])
    ce = (ces[0] + ces[1]) + (ces[2] + ces[3])

    validg = cg[7:8, :]
    kf = cg[5:6, :]
    sr = cg[6:7, :]
    kfT = jnp.transpose(kf)
    srT = jnp.transpose(sr)
    vT = jnp.transpose(validg)
    iot = lane_ff
    iotT = jnp.transpose(iot)
    beats = (vT > 0.0) & (
        (kfT > kf) | ((kfT == kf) & ((srT > sr) | ((srT == sr)
                                                   & (iotT < iot)))))
    rank_g = jnp.sum(jnp.where(beats, 1.0, 0.0), axis=0, keepdims=True)
    g11 = jnp.sum(gm, keepdims=True)

    iota_t = jax.lax.broadcasted_iota(jnp.int32, (DETS, 1), 0).astype(f32)
    ohg = jnp.where((rank_g == iota_t) & (validg > 0.0), 1.0, 0.0)
    rank_e = g11 + lane_ff
    ohe = jnp.where((rank_e == iota_t) & (ce[7:8, :] > 0.0), 1.0, 0.0)
    outg = jax.lax.dot_general(
        ohg, jnp.transpose(cg), (((1,), (0,)), ((), ())),
        preferred_element_type=f32,
        precision=jax.lax.Precision.HIGHEST)
    oute = jax.lax.dot_general(
        ohe, jnp.transpose(ce), (((1,), (0,)), ((), ())),
        preferred_element_type=f32,
        precision=jax.lax.Precision.HIGHEST)
    outfull = outg + oute
    out_ref[...] = outfull[:, 0:5]


@jax.jit
def kernel(boxes, scores):
    pad = NP - N
    bpad = jnp.pad(boxes, ((0, pad), (0, 0)))
    spad = jnp.pad(scores, (0, pad), constant_values=-1.0)
    bj = bpad.T.reshape(4, ROWS, 128)
    sjm = spad.reshape(ROWS, 128)

    f32 = jnp.float32
    out = pl.pallas_call(
        _nms_kernel,
        out_shape=jax.ShapeDtypeStruct((DETS, 5), f32),
        scratch_shapes=(
            [pltpu.VMEM((ROWS, 128), f32) for _ in range(5)]
            + [pltpu.VMEM((ROWS, 128), f32),
               pltpu.VMEM((128, ROWS), f32),
]),
    )(bj, sjm)
    return out
